# baseline (speedup 1.0000x reference)
import math

import jax
import jax.numpy as jnp
from jax import lax
from jax.experimental import pallas as pl
from jax.experimental.pallas import tpu as pltpu
from jax.experimental.pallas import tpu_sc as plsc

_EMB = 64
_SCALE = math.sqrt(_EMB)
_NC, _NS = 2, 16
_NW = _NC * _NS
_C = 512


def _sc_body(idx_hbm, table_hbm, out_hbm, idx_all, rows, gsem0, gsem1):
    wid = lax.axis_index("s") * _NC + lax.axis_index("c")
    n = idx_hbm.shape[0]
    b_per_w = n // _NW
    base = wid * b_per_w
    nchunk = b_per_w // _C
    gsems = (gsem0, gsem1)

    pltpu.sync_copy(idx_hbm.at[pl.ds(base, b_per_w)], idx_all)

    def gather(g, b):
        pltpu.async_copy(
            table_hbm.at[idx_all.at[pl.ds(g * _C, _C)]], rows.at[b], gsems[b])

    for b in range(2):
        gather(b, b)

    def pair(p, carry):
        for b in range(2):
            g = 2 * p + b
            pltpu.make_async_copy(
                table_hbm.at[idx_all.at[pl.ds(0, _C)]], rows.at[b],
                gsems[b]).wait()

            def scale(r, c2):
                for j in range(_EMB // 16):
                    sl = pl.ds(j * 16, 16)
                    rows[b, r, sl] = rows[b, r, sl] * _SCALE
                return c2

            lax.fori_loop(0, _C, scale, 0, unroll=4)
            pltpu.sync_copy(rows.at[b], out_hbm.at[pl.ds(base + g * _C, _C)])

            @pl.when(g + 2 < nchunk)
            def _():
                gather(g + 2, b)
        return carry

    lax.fori_loop(0, nchunk // 2, pair, 0)


def kernel(tokens, table):
    b, l = tokens.shape
    idx = tokens.reshape(-1).astype(jnp.int32)
    n = idx.shape[0]
    mesh = plsc.VectorSubcoreMesh(
        core_axis_name="c", subcore_axis_name="s",
        num_cores=_NC, num_subcores=_NS,
    )
    run = pl.kernel(
        _sc_body,
        out_type=jax.ShapeDtypeStruct((n, _EMB), jnp.float32),
        mesh=mesh,
        scratch_types=[
            pltpu.VMEM((n // _NW,), jnp.int32),
            pltpu.VMEM((2, _C, _EMB), jnp.float32),
            pltpu.SemaphoreType.DMA,
            pltpu.SemaphoreType.DMA,
        ],
        compiler_params=pltpu.CompilerParams(use_tc_tiling_on_sc=False),
    )
    out = run(idx, table)
    return out.reshape(b, l, _EMB)
